# baseline (speedup 1.0000x reference)
import jax
import jax.numpy as jnp
from jax.experimental import pallas as pl
from jax.experimental.pallas import tpu as pltpu

G = 2
NQ = 4
CS = 4096
D = 128
DG = D // G
BT = 16 * 2048
TB = 1024
NB = BT // TB


def _bf(v):
    return v.astype(jnp.bfloat16)


def _full(spec):
    return pl.BlockSpec(spec, lambda *_: tuple(0 for _ in spec))


def _planes_cat(cb):
    def trunc16(v):
        bits = jax.lax.bitcast_convert_type(v, jnp.int32)
        return jax.lax.bitcast_convert_type(
            bits & jnp.int32(-65536), jnp.float32)
    hi = trunc16(cb)
    r1 = cb - hi
    mid = trunc16(r1)
    lo = r1 - mid
    return jnp.concatenate([_bf(hi), _bf(mid), _bf(lo)], axis=-1)


def _block_diag2(w):
    z = jnp.zeros((DG, DG), w.dtype)
    return jnp.block([[w[0], z], [z, w[1]]])


def _assemble_f32(planes_f32):
    return ((planes_f32[:, 0 * DG:1 * DG]
             + planes_f32[:, 1 * DG:2 * DG])
            + planes_f32[:, 2 * DG:3 * DG])


def _vq_body(x_ref, Winbd_ref, bin_ref, Woutbd_ref, bout_ref,
             cbTbd_ref, pcat_ref, cbn_ref,
             quant_ref, recon_ref, dsum_ref):
    x = x_ref[...]
    iota = jax.lax.broadcasted_iota(jnp.int32, (TB, CS), 1)

    xin = jnp.dot(_bf(x), _bf(Winbd_ref[...]),
                  preferred_element_type=jnp.float32) + bin_ref[...][None, :]
    r = xin

    dsum = jnp.zeros((), jnp.float32)
    for q in range(NQ):
        ab = jnp.dot(_bf(r), cbTbd_ref[q],
                     preferred_element_type=jnp.float32)
        quants = []
        for g in range(G):
            d = ab[:, g * CS:(g + 1) * CS] \
                + cbn_ref[q, g * CS:(g + 1) * CS][None, :]
            idx = jnp.argmin(d, axis=-1)
            oh = (iota == idx[:, None]).astype(jnp.bfloat16)
            planes_f32 = jnp.dot(oh, pcat_ref[g, q],
                                 preferred_element_type=jnp.float32)
            quants.append(_assemble_f32(planes_f32))
        quant = jnp.concatenate(quants, axis=-1)
        r = r - quant
        dsum = dsum + jnp.sum(r * r)

    qout = xin - r
    quantized = jnp.dot(_bf(qout), _bf(Woutbd_ref[...]),
                        preferred_element_type=jnp.float32) \
        + bout_ref[...][None, :]
    quant_ref[...] = quantized
    diff = x - quantized
    recon_ref[...] = jnp.sum(diff * diff, axis=-1) * (1.0 / D)
    dsum_ref[...] = dsum.reshape(1, 1, 1)


def _loss_body(dsum_ref, recon_ref, loss_ref, cm_ref):
    cm = jnp.sum(dsum_ref[...]) * (1.0 / (G * NQ * BT * DG))
    loss_ref[...] = recon_ref[...] + cm
    cm_ref[...] = cm.reshape(1, 1)


def _pipeline(x, W_in, b_in, W_out, b_out, codebooks, interpret=False):
    Bb, Tt, _ = x.shape
    xf = x.reshape(BT, D)

    cbT = (codebooks * -2.0).transpose(0, 1, 3, 2).astype(jnp.bfloat16)
    zpad = jnp.zeros((NQ, DG, CS), jnp.bfloat16)
    top = jnp.concatenate([cbT[0], zpad], axis=2)
    bot = jnp.concatenate([zpad, cbT[1]], axis=2)
    cbTbd = jnp.concatenate([top, bot], axis=1)
    pcat = _planes_cat(codebooks)
    cbn = jnp.sum(codebooks * codebooks, axis=-1)
    cbn_cat = jnp.concatenate([cbn[0], cbn[1]], axis=-1)
    Winbd = _block_diag2(W_in)
    Woutbd = _block_diag2(W_out)
    bin_cat = b_in.reshape(D)
    bout_cat = b_out.reshape(D)

    quantized, recon, dsum = pl.pallas_call(
        _vq_body,
        grid=(NB,),
        in_specs=[
            pl.BlockSpec((TB, D), lambda i: (i, 0)),
            _full((D, D)),
            _full((D,)),
            _full((D, D)),
            _full((D,)),
            _full((NQ, D, 2 * CS)),
            _full((G, NQ, CS, 3 * DG)),
            _full((NQ, 2 * CS)),
        ],
        out_specs=[
            pl.BlockSpec((TB, D), lambda i: (i, 0)),
            pl.BlockSpec((TB,), lambda i: (i,)),
            pl.BlockSpec((1, 1, 1), lambda i: (i, 0, 0)),
        ],
        out_shape=[
            jax.ShapeDtypeStruct((BT, D), jnp.float32),
            jax.ShapeDtypeStruct((BT,), jnp.float32),
            jax.ShapeDtypeStruct((NB, 1, 1), jnp.float32),
        ],
        compiler_params=pltpu.CompilerParams(
            dimension_semantics=("parallel",)),
        interpret=interpret,
    )(xf, Winbd, bin_cat, Woutbd, bout_cat, cbTbd, pcat, cbn_cat)

    loss, cm = pl.pallas_call(
        _loss_body,
        grid=(NB,),
        in_specs=[
            _full((NB, 1, 1)),
            pl.BlockSpec((TB,), lambda i: (i,)),
        ],
        out_specs=[
            pl.BlockSpec((TB,), lambda i: (i,)),
            pl.BlockSpec((1, 1), lambda i: (0, 0)),
        ],
        out_shape=[
            jax.ShapeDtypeStruct((BT,), jnp.float32),
            jax.ShapeDtypeStruct((1, 1), jnp.float32),
        ],
        compiler_params=pltpu.CompilerParams(
            dimension_semantics=("arbitrary",)),
        interpret=interpret,
    )(dsum, recon)

    return (quantized.reshape(Bb, Tt, D), loss.reshape(Bb, Tt),
            cm.reshape(()), recon.reshape(Bb, Tt))


def kernel(x, W_in, b_in, W_out, b_out, codebooks):
    return _pipeline(x, W_in, b_in, W_out, b_out, codebooks)


# speedup vs baseline: 1.0656x; 1.0656x over previous
import jax
import jax.numpy as jnp
from jax.experimental import pallas as pl
from jax.experimental.pallas import tpu as pltpu

G = 2
NQ = 4
CS = 4096
D = 128
DG = D // G
BT = 16 * 2048
TB = 256
NB = BT // TB


def _bf(v):
    return v.astype(jnp.bfloat16)


def _full(spec):
    return pl.BlockSpec(spec, lambda *_: tuple(0 for _ in spec))


def _planes_cat(cb):
    def trunc16(v):
        bits = jax.lax.bitcast_convert_type(v, jnp.int32)
        return jax.lax.bitcast_convert_type(
            bits & jnp.int32(-65536), jnp.float32)
    hi = trunc16(cb)
    r1 = cb - hi
    mid = trunc16(r1)
    lo = r1 - mid
    return jnp.concatenate([_bf(hi), _bf(mid), _bf(lo)], axis=-1)


def _block_diag2(w):
    z = jnp.zeros((DG, DG), w.dtype)
    return jnp.block([[w[0], z], [z, w[1]]])


def _assemble_f32(planes_f32):
    return ((planes_f32[:, 0 * DG:1 * DG]
             + planes_f32[:, 1 * DG:2 * DG])
            + planes_f32[:, 2 * DG:3 * DG])


def _vq_body(x_ref, Winbd_ref, bin_ref, Woutbd_ref, bout_ref,
             cbTbd_ref, pcat_ref, cbn_ref,
             quant_ref, recon_ref, dsum_ref):
    x = x_ref[...]
    iota = jax.lax.broadcasted_iota(jnp.int32, (TB, CS), 1)

    xin = jnp.dot(_bf(x), _bf(Winbd_ref[...]),
                  preferred_element_type=jnp.float32) + bin_ref[...][None, :]
    r = xin

    dsum = jnp.zeros((), jnp.float32)
    for q in range(NQ):
        ab = jnp.dot(_bf(r), cbTbd_ref[q],
                     preferred_element_type=jnp.float32)
        quants = []
        for g in range(G):
            d = ab[:, g * CS:(g + 1) * CS] \
                + cbn_ref[q, g * CS:(g + 1) * CS][None, :]
            idx = jnp.argmin(d, axis=-1)
            oh = (iota == idx[:, None]).astype(jnp.bfloat16)
            planes_f32 = jnp.dot(oh, pcat_ref[g, q],
                                 preferred_element_type=jnp.float32)
            quants.append(_assemble_f32(planes_f32))
        quant = jnp.concatenate(quants, axis=-1)
        r = r - quant
        dsum = dsum + jnp.sum(r * r)

    qout = xin - r
    quantized = jnp.dot(_bf(qout), _bf(Woutbd_ref[...]),
                        preferred_element_type=jnp.float32) \
        + bout_ref[...][None, :]
    quant_ref[...] = quantized
    diff = x - quantized
    recon_ref[...] = jnp.sum(diff * diff, axis=-1) * (1.0 / D)
    dsum_ref[...] = dsum.reshape(1, 1, 1)


def _loss_body(dsum_ref, recon_ref, loss_ref, cm_ref):
    cm = jnp.sum(dsum_ref[...]) * (1.0 / (G * NQ * BT * DG))
    loss_ref[...] = recon_ref[...] + cm
    cm_ref[...] = cm.reshape(1, 1)


def _pipeline(x, W_in, b_in, W_out, b_out, codebooks, interpret=False):
    Bb, Tt, _ = x.shape
    xf = x.reshape(BT, D)

    cbT = (codebooks * -2.0).transpose(0, 1, 3, 2).astype(jnp.bfloat16)
    zpad = jnp.zeros((NQ, DG, CS), jnp.bfloat16)
    top = jnp.concatenate([cbT[0], zpad], axis=2)
    bot = jnp.concatenate([zpad, cbT[1]], axis=2)
    cbTbd = jnp.concatenate([top, bot], axis=1)
    pcat = _planes_cat(codebooks)
    cbn = jnp.sum(codebooks * codebooks, axis=-1)
    cbn_cat = jnp.concatenate([cbn[0], cbn[1]], axis=-1)
    Winbd = _block_diag2(W_in)
    Woutbd = _block_diag2(W_out)
    bin_cat = b_in.reshape(D)
    bout_cat = b_out.reshape(D)

    quantized, recon, dsum = pl.pallas_call(
        _vq_body,
        grid=(NB,),
        in_specs=[
            pl.BlockSpec((TB, D), lambda i: (i, 0)),
            _full((D, D)),
            _full((D,)),
            _full((D, D)),
            _full((D,)),
            _full((NQ, D, 2 * CS)),
            _full((G, NQ, CS, 3 * DG)),
            _full((NQ, 2 * CS)),
        ],
        out_specs=[
            pl.BlockSpec((TB, D), lambda i: (i, 0)),
            pl.BlockSpec((TB,), lambda i: (i,)),
            pl.BlockSpec((1, 1, 1), lambda i: (i, 0, 0)),
        ],
        out_shape=[
            jax.ShapeDtypeStruct((BT, D), jnp.float32),
            jax.ShapeDtypeStruct((BT,), jnp.float32),
            jax.ShapeDtypeStruct((NB, 1, 1), jnp.float32),
        ],
        compiler_params=pltpu.CompilerParams(
            dimension_semantics=("parallel",)),
        interpret=interpret,
    )(xf, Winbd, bin_cat, Woutbd, bout_cat, cbTbd, pcat, cbn_cat)

    loss, cm = pl.pallas_call(
        _loss_body,
        grid=(NB,),
        in_specs=[
            _full((NB, 1, 1)),
            pl.BlockSpec((TB,), lambda i: (i,)),
        ],
        out_specs=[
            pl.BlockSpec((TB,), lambda i: (i,)),
            pl.BlockSpec((1, 1), lambda i: (0, 0)),
        ],
        out_shape=[
            jax.ShapeDtypeStruct((BT,), jnp.float32),
            jax.ShapeDtypeStruct((1, 1), jnp.float32),
        ],
        compiler_params=pltpu.CompilerParams(
            dimension_semantics=("arbitrary",)),
        interpret=interpret,
    )(dsum, recon)

    return (quantized.reshape(Bb, Tt, D), loss.reshape(Bb, Tt),
            cm.reshape(()), recon.reshape(Bb, Tt))


def kernel(x, W_in, b_in, W_out, b_out, codebooks):
    return _pipeline(x, W_in, b_in, W_out, b_out, codebooks)
